# Initial kernel scaffold; baseline (speedup 1.0000x reference)
import jax
import jax.numpy as jnp
from jax.experimental import pallas as pl


def kernel(x, edge_attr, params, edge_index):
    raise NotImplementedError("write your pallas kernel here")


# speedup vs baseline: 1.8976x; 1.8976x over previous
import functools

import jax
import jax.numpy as jnp
from jax import lax
from jax.experimental import pallas as pl
from jax.experimental.pallas import tpu as pltpu
from jax.experimental.pallas import tpu_sc as plsc

F32 = jnp.float32


def _dotT(a, w):
    return lax.dot_general(a, w, (((1,), (1,)), ((), ())),
                           preferred_element_type=F32)


def _mlp2_node_kernel(x_ref, w1_ref, b1_ref, w2_ref, b2_ref, o_ref):
    h1 = jnp.maximum(_dotT(x_ref[...], w1_ref[...]) + b1_ref[...], 0.0)
    o_ref[...] = _dotT(h1, w2_ref[...]) + b2_ref[...]


def _tc_mlp2(p, x):
    n = x.shape[0]
    dout = p["W2"].shape[0]
    return pl.pallas_call(
        _mlp2_node_kernel,
        out_shape=jax.ShapeDtypeStruct((n, dout), F32),
    )(x, p["W1"], p["b1"].reshape(1, -1), p["W2"], p["b2"].reshape(1, -1))


def _edge_kernel(ea_ref, w1e_ref, b1e_ref, w2e_ref, b2e_ref,
                 w1g_ref, b1g_ref, w2g_ref, b2g_ref, ee_ref, g_ref):
    ea = ea_ref[...]
    h1 = jnp.maximum(_dotT(ea, w1e_ref[...]) + b1e_ref[...], 0.0)
    ee_ref[...] = _dotT(h1, w2e_ref[...]) + b2e_ref[...]
    hg = jnp.maximum(_dotT(ea, w1g_ref[...]) + b1g_ref[...], 0.0)
    gr = jnp.sum(hg * w2g_ref[...], axis=1, keepdims=True) + b2g_ref[...]
    g_ref[...] = jax.nn.sigmoid(gr)


def _tc_edge(lp, edge_attr, be=4000):
    e, ed = edge_attr.shape
    h = lp["edge"]["W2"].shape[0]
    grid = e // be
    ep = lp["edge"]
    gp = lp["gate"]
    ee, g = pl.pallas_call(
        _edge_kernel,
        grid=(grid,),
        in_specs=[
            pl.BlockSpec((be, ed), lambda i: (i, 0)),
            pl.BlockSpec((h, ed), lambda i: (0, 0)),
            pl.BlockSpec((1, h), lambda i: (0, 0)),
            pl.BlockSpec((h, h), lambda i: (0, 0)),
            pl.BlockSpec((1, h), lambda i: (0, 0)),
            pl.BlockSpec((h, ed), lambda i: (0, 0)),
            pl.BlockSpec((1, h), lambda i: (0, 0)),
            pl.BlockSpec((1, h), lambda i: (0, 0)),
            pl.BlockSpec((1, 1), lambda i: (0, 0)),
        ],
        out_specs=[
            pl.BlockSpec((be, h), lambda i: (i, 0)),
            pl.BlockSpec((be, 1), lambda i: (i, 0)),
        ],
        out_shape=[
            jax.ShapeDtypeStruct((e, h), F32),
            jax.ShapeDtypeStruct((e, 1), F32),
        ],
    )(edge_attr, ep["W1"], ep["b1"].reshape(1, -1), ep["W2"],
      ep["b2"].reshape(1, -1), gp["W1"], gp["b1"].reshape(1, -1),
      gp["W2"], gp["b2"].reshape(1, 1))
    return ee, g.reshape(e)


def _lin_kernel(x_ref, w_ref, b_ref, o_ref):
    o_ref[...] = _dotT(x_ref[...], w_ref[...]) + b_ref[...]


def _tc_lin(p, x):
    n = x.shape[0]
    dout = p["W"].shape[0]
    return pl.pallas_call(
        _lin_kernel,
        out_shape=jax.ShapeDtypeStruct((n, dout), F32),
    )(x, p["W"], p["b"].reshape(1, -1))


def _update_kernel(h_ref, sums_ref, cnt_ref, w1a_ref, w1b_ref, b1_ref,
                   w2_ref, b2_ref, lng_ref, lnb_ref, o_ref):
    s = sums_ref[0] + sums_ref[1]
    cnt = cnt_ref[0, :, 0:1] + cnt_ref[1, :, 0:1]
    agg = s / jnp.maximum(cnt, 1.0)
    hh = h_ref[...]
    z = _dotT(hh, w1a_ref[...]) + _dotT(agg, w1b_ref[...]) + b1_ref[...]
    z = jnp.maximum(z, 0.0)
    upd = _dotT(z, w2_ref[...]) + b2_ref[...]
    y = hh + upd
    m = jnp.mean(y, axis=-1, keepdims=True)
    v = jnp.mean((y - m) ** 2, axis=-1, keepdims=True)
    o_ref[...] = (y - m) / jnp.sqrt(v + 1e-5) * lng_ref[...] + lnb_ref[...]


def _tc_update(lp, h, sums_parts, cnt_parts):
    n, hd = h.shape
    up = lp["upd"]
    w1 = up["W1"]
    w1a = w1[:, :hd]
    w1b = w1[:, hd:]
    return pl.pallas_call(
        _update_kernel,
        out_shape=jax.ShapeDtypeStruct((n, hd), F32),
    )(h, sums_parts, cnt_parts, w1a, w1b, up["b1"].reshape(1, -1),
      up["W2"], up["b2"].reshape(1, -1), lp["ln_g"].reshape(1, -1),
      lp["ln_b"].reshape(1, -1))


def _head_kernel(h_ref, w1_ref, b1_ref, w2_ref, b2_ref, sc_ref, o_ref):
    h1 = jnp.maximum(_dotT(h_ref[...], w1_ref[...]) + b1_ref[...], 0.0)
    raw = (_dotT(h1, w2_ref[...]) + b2_ref[...]) * sc_ref[0, 0]
    o_ref[...] = 1.5 * jnp.tanh(raw)


def _tc_head(p, scale, h):
    n = h.shape[0]
    a = p["W2"].shape[0]
    return pl.pallas_call(
        _head_kernel,
        out_shape=jax.ShapeDtypeStruct((n, a), F32),
    )(h, p["W1"], p["b1"].reshape(1, -1), p["W2"], p["b2"].reshape(1, -1),
      scale.reshape(1, 1))


CH = 128


def _sc_aggregate(hn, ee, g, src, dst):
    n, hd = hn.shape
    e = ee.shape[0]
    nchunk = e // CH
    nzw = 10
    rps = n // nzw

    mesh = plsc.VectorSubcoreMesh(core_axis_name="c", subcore_axis_name="s")

    @functools.partial(
        pl.kernel, mesh=mesh,
        out_type=jax.ShapeDtypeStruct((2, n, hd), F32),
        scratch_types=[
            pltpu.VMEM((CH, hd), F32),
            pltpu.VMEM((CH, hd), F32),
            pltpu.VMEM((CH,), jnp.int32),
            pltpu.VMEM((CH,), jnp.int32),
            pltpu.VMEM((CH,), F32),
            pltpu.VMEM((128, 128), F32),
            pltpu.VMEM_SHARED((n, hd), F32),
        ])
    def kern(hn_hbm, ee_hbm, g_hbm, src_hbm, dst_hbm, sums_out,
             ee_v, rows_v, src_v, dst_v, g_v, z128, sums_sh):
        cid = lax.axis_index("c")
        sid = lax.axis_index("s")

        if True:
            def zrow(i, c):
                for j in range(8):
                    z128[i, pl.ds(j * 16, 16)] = jnp.zeros((16,), F32)
                return c
            lax.fori_loop(0, 128, zrow, 0)
            base = sid * rps
            nfull = rps // 128
            tail = rps % 128

            @pl.when(sid < nzw)
            def _init():
                for k in range(nfull):
                    pltpu.sync_copy(z128,
                                    sums_sh.at[pl.ds(base + k * 128, 128)])
                if tail:
                    lo = base + nfull * 128
                    pltpu.sync_copy(z128.at[pl.ds(0, tail)],
                                    sums_sh.at[pl.ds(lo, tail)])
            plsc.subcore_barrier()

            wid = sid * 2 + cid
            ntrip = (nchunk - wid + 31) // 32

            def chunk(t, c):
                b0 = (wid + t * 32) * CH
                pltpu.sync_copy(src_hbm.at[pl.ds(b0, CH)], src_v)
                pltpu.sync_copy(dst_hbm.at[pl.ds(b0, CH)], dst_v)
                pltpu.sync_copy(g_hbm.at[pl.ds(b0, CH)], g_v)
                pltpu.sync_copy(ee_hbm.at[pl.ds(b0, CH)], ee_v)
                pltpu.sync_copy(hn_hbm.at[src_v], rows_v)

                def rowfn(gg, cc):
                    gvec = g_v[pl.ds(gg * 16, 16)]

                    def row(r, c2):
                        i = gg * 16 + r
                        gs = gvec[jnp.full((16,), r, jnp.int32)]
                        for j in range(hd // 16):
                            sl = pl.ds(j * 16, 16)
                            ee_v[i, sl] = gs * (ee_v[i, sl] + rows_v[i, sl])
                        return c2
                    lax.fori_loop(0, 16, row, 0)
                    return cc
                lax.fori_loop(0, CH // 16, rowfn, 0)

                pltpu.sync_copy(ee_v, sums_sh.at[dst_v], add=True)
                return c
            lax.fori_loop(0, ntrip, chunk, 0)

            plsc.subcore_barrier()

            @pl.when(sid < nzw)
            def _drain():
                for k in range(nfull):
                    lo = base + k * 128
                    pltpu.sync_copy(sums_sh.at[pl.ds(lo, 128)], z128)
                    pltpu.sync_copy(z128, sums_out.at[cid, pl.ds(lo, 128)])
                if tail:
                    lo = base + nfull * 128
                    pltpu.sync_copy(sums_sh.at[pl.ds(lo, tail)],
                                    z128.at[pl.ds(0, tail)])
                    pltpu.sync_copy(z128.at[pl.ds(0, tail)],
                                    sums_out.at[cid, pl.ds(lo, tail)])

    return kern(hn, ee, g, src, dst)


def _sc_counts(dst, n, hd):
    e = dst.shape[0]
    nchunk = e // CH
    nzw = 10
    rps = n // nzw

    mesh = plsc.VectorSubcoreMesh(core_axis_name="c", subcore_axis_name="s")

    @functools.partial(
        pl.kernel, mesh=mesh,
        out_type=jax.ShapeDtypeStruct((2, n, hd), F32),
        scratch_types=[
            pltpu.VMEM((CH, hd), F32),
            pltpu.VMEM((CH,), jnp.int32),
            pltpu.VMEM((128, 128), F32),
            pltpu.VMEM_SHARED((n, hd), F32),
        ])
    def kern(dst_hbm, cnt_out, ones_v, dst_v, z128, cnt_sh):
        cid = lax.axis_index("c")
        sid = lax.axis_index("s")

        if True:
            def zrow(i, c):
                for j in range(8):
                    z128[i, pl.ds(j * 16, 16)] = jnp.zeros((16,), F32)
                    ones_v[i, pl.ds(j * 16, 16)] = jnp.full((16,), 1.0, F32)
                return c
            lax.fori_loop(0, 128, zrow, 0)
            base = sid * rps
            nfull = rps // 128
            tail = rps % 128

            @pl.when(sid < nzw)
            def _init():
                for k in range(nfull):
                    pltpu.sync_copy(z128,
                                    cnt_sh.at[pl.ds(base + k * 128, 128)])
                if tail:
                    lo = base + nfull * 128
                    pltpu.sync_copy(z128.at[pl.ds(0, tail)],
                                    cnt_sh.at[pl.ds(lo, tail)])
            plsc.subcore_barrier()

            wid = sid * 2 + cid
            ntrip = (nchunk - wid + 31) // 32

            def chunk(t, c):
                b0 = (wid + t * 32) * CH
                pltpu.sync_copy(dst_hbm.at[pl.ds(b0, CH)], dst_v)
                pltpu.sync_copy(ones_v, cnt_sh.at[dst_v], add=True)
                return c
            lax.fori_loop(0, ntrip, chunk, 0)

            plsc.subcore_barrier()

            @pl.when(sid < nzw)
            def _drain():
                for k in range(nfull):
                    lo = base + k * 128
                    pltpu.sync_copy(cnt_sh.at[pl.ds(lo, 128)], z128)
                    pltpu.sync_copy(z128, cnt_out.at[cid, pl.ds(lo, 128)])
                if tail:
                    lo = base + nfull * 128
                    pltpu.sync_copy(cnt_sh.at[pl.ds(lo, tail)],
                                    z128.at[pl.ds(0, tail)])
                    pltpu.sync_copy(z128.at[pl.ds(0, tail)],
                                    cnt_out.at[cid, pl.ds(lo, tail)])

    return kern(dst)


def kernel(x, edge_attr, params, edge_index):
    src = edge_index[0].astype(jnp.int32)
    dst = edge_index[1].astype(jnp.int32)

    h = _tc_mlp2(params["enc"], x)
    cnt_parts = _sc_counts(dst, x.shape[0], 128)
    for lp in params["layers"]:
        ee, g = _tc_edge(lp, edge_attr)
        hn = _tc_lin(lp["neigh"], h)
        sums_parts = _sc_aggregate(hn, ee, g, src, dst)
        h = _tc_update(lp, h, sums_parts, cnt_parts)
    return _tc_head(params["head"], params["scale"], h)


# speedup vs baseline: 2.2575x; 1.1897x over previous
import functools

import jax
import jax.numpy as jnp
from jax import lax
from jax.experimental import pallas as pl
from jax.experimental.pallas import tpu as pltpu
from jax.experimental.pallas import tpu_sc as plsc

F32 = jnp.float32


def _dotT(a, w):
    return lax.dot_general(a, w, (((1,), (1,)), ((), ())),
                           preferred_element_type=F32)


def _mlp2_node_kernel(x_ref, w1_ref, b1_ref, w2_ref, b2_ref, o_ref):
    h1 = jnp.maximum(_dotT(x_ref[...], w1_ref[...]) + b1_ref[...], 0.0)
    o_ref[...] = _dotT(h1, w2_ref[...]) + b2_ref[...]


def _tc_mlp2(p, x):
    n = x.shape[0]
    dout = p["W2"].shape[0]
    return pl.pallas_call(
        _mlp2_node_kernel,
        out_shape=jax.ShapeDtypeStruct((n, dout), F32),
    )(x, p["W1"], p["b1"].reshape(1, -1), p["W2"], p["b2"].reshape(1, -1))


def _edge_kernel(ea_ref, w1e_ref, b1e_ref, w2e_ref, b2e_ref,
                 w1g_ref, b1g_ref, w2g_ref, b2g_ref, ee_ref, g_ref):
    ea = ea_ref[...]
    h1 = jnp.maximum(_dotT(ea, w1e_ref[...]) + b1e_ref[...], 0.0)
    ee = _dotT(h1, w2e_ref[...]) + b2e_ref[...]
    hg = jnp.maximum(_dotT(ea, w1g_ref[...]) + b1g_ref[...], 0.0)
    gr = jnp.sum(hg * w2g_ref[...], axis=1, keepdims=True) + b2g_ref[...]
    g = jax.nn.sigmoid(gr)
    g_ref[...] = g
    ee_ref[...] = ee * g


def _tc_edge(lp, edge_attr, be=4000):
    e, ed = edge_attr.shape
    h = lp["edge"]["W2"].shape[0]
    grid = e // be
    ep = lp["edge"]
    gp = lp["gate"]
    ee, g = pl.pallas_call(
        _edge_kernel,
        grid=(grid,),
        in_specs=[
            pl.BlockSpec((be, ed), lambda i: (i, 0)),
            pl.BlockSpec((h, ed), lambda i: (0, 0)),
            pl.BlockSpec((1, h), lambda i: (0, 0)),
            pl.BlockSpec((h, h), lambda i: (0, 0)),
            pl.BlockSpec((1, h), lambda i: (0, 0)),
            pl.BlockSpec((h, ed), lambda i: (0, 0)),
            pl.BlockSpec((1, h), lambda i: (0, 0)),
            pl.BlockSpec((1, h), lambda i: (0, 0)),
            pl.BlockSpec((1, 1), lambda i: (0, 0)),
        ],
        out_specs=[
            pl.BlockSpec((be, h), lambda i: (i, 0)),
            pl.BlockSpec((be, 1), lambda i: (i, 0)),
        ],
        out_shape=[
            jax.ShapeDtypeStruct((e, h), F32),
            jax.ShapeDtypeStruct((e, 1), F32),
        ],
    )(edge_attr, ep["W1"], ep["b1"].reshape(1, -1), ep["W2"],
      ep["b2"].reshape(1, -1), gp["W1"], gp["b1"].reshape(1, -1),
      gp["W2"], gp["b2"].reshape(1, 1))
    return ee, g.reshape(e)


def _lin_kernel(x_ref, w_ref, b_ref, o_ref):
    o_ref[...] = _dotT(x_ref[...], w_ref[...]) + b_ref[...]


def _tc_lin(p, x):
    n = x.shape[0]
    dout = p["W"].shape[0]
    return pl.pallas_call(
        _lin_kernel,
        out_shape=jax.ShapeDtypeStruct((n, dout), F32),
    )(x, p["W"], p["b"].reshape(1, -1))


def _update_kernel(h_ref, sums_ref, cnt_ref, w1a_ref, w1b_ref, b1_ref,
                   w2_ref, b2_ref, lng_ref, lnb_ref, o_ref):
    s = sums_ref[0] + sums_ref[1]
    cnt = cnt_ref[0, :, 0:1] + cnt_ref[1, :, 0:1]
    agg = s / jnp.maximum(cnt, 1.0)
    hh = h_ref[...]
    z = _dotT(hh, w1a_ref[...]) + _dotT(agg, w1b_ref[...]) + b1_ref[...]
    z = jnp.maximum(z, 0.0)
    upd = _dotT(z, w2_ref[...]) + b2_ref[...]
    y = hh + upd
    m = jnp.mean(y, axis=-1, keepdims=True)
    v = jnp.mean((y - m) ** 2, axis=-1, keepdims=True)
    o_ref[...] = (y - m) / jnp.sqrt(v + 1e-5) * lng_ref[...] + lnb_ref[...]


def _tc_update(lp, h, sums_parts, cnt_parts):
    n, hd = h.shape
    up = lp["upd"]
    w1 = up["W1"]
    w1a = w1[:, :hd]
    w1b = w1[:, hd:]
    return pl.pallas_call(
        _update_kernel,
        out_shape=jax.ShapeDtypeStruct((n, hd), F32),
    )(h, sums_parts, cnt_parts, w1a, w1b, up["b1"].reshape(1, -1),
      up["W2"], up["b2"].reshape(1, -1), lp["ln_g"].reshape(1, -1),
      lp["ln_b"].reshape(1, -1))


def _head_kernel(h_ref, w1_ref, b1_ref, w2_ref, b2_ref, sc_ref, o_ref):
    h1 = jnp.maximum(_dotT(h_ref[...], w1_ref[...]) + b1_ref[...], 0.0)
    raw = (_dotT(h1, w2_ref[...]) + b2_ref[...]) * sc_ref[0, 0]
    o_ref[...] = 1.5 * jnp.tanh(raw)


def _tc_head(p, scale, h):
    n = h.shape[0]
    a = p["W2"].shape[0]
    return pl.pallas_call(
        _head_kernel,
        out_shape=jax.ShapeDtypeStruct((n, a), F32),
    )(h, p["W1"], p["b1"].reshape(1, -1), p["W2"], p["b2"].reshape(1, -1),
      scale.reshape(1, 1))


CH = 128
CHA = 64


def _sc_aggregate(hn, me, g, src, dst):
    n, hd = hn.shape
    e = me.shape[0]
    nchunk = e // CHA
    nzw = 10
    rps = n // nzw
    maxtrip = (nchunk + 31) // 32
    npass = -(-maxtrip // 4) * 4

    mesh = plsc.VectorSubcoreMesh(core_axis_name="c", subcore_axis_name="s")

    @functools.partial(
        pl.kernel, mesh=mesh,
        out_type=jax.ShapeDtypeStruct((2, n, hd), F32),
        scratch_types=[
            pltpu.VMEM((CHA, hd), F32),
            pltpu.VMEM((CHA, hd), F32),
            pltpu.VMEM((CHA, hd), F32),
            pltpu.VMEM((CHA, hd), F32),
            pltpu.VMEM((4, CHA), jnp.int32),
            pltpu.VMEM((4, CHA), jnp.int32),
            pltpu.VMEM((4, CHA), F32),
            pltpu.VMEM((128, 128), F32),
            pltpu.SemaphoreType.DMA,
            pltpu.SemaphoreType.DMA,
            pltpu.SemaphoreType.DMA,
            pltpu.SemaphoreType.DMA,
            pltpu.SemaphoreType.DMA,
            pltpu.SemaphoreType.DMA,
            pltpu.SemaphoreType.DMA,
            pltpu.SemaphoreType.DMA,
            pltpu.VMEM_SHARED((n, hd), F32),
        ])
    def kern(hn_hbm, me_hbm, g_hbm, src_hbm, dst_hbm, sums_out,
             me0, me1, rw0, rw1, sidx, didx, gslot, z128,
             l0, l1, l2, l3, ga0, ga1, sc0, sc1, sums_sh):
        cid = lax.axis_index("c")
        sid = lax.axis_index("s")
        mes = (me0, me1)
        rws = (rw0, rw1)
        lsems = (l0, l1, l2, l3)
        gsems = (ga0, ga1)
        ssems = (sc0, sc1)

        def zrow(i, c):
            for j in range(8):
                z128[i, pl.ds(j * 16, 16)] = jnp.zeros((16,), F32)
            return c
        lax.fori_loop(0, 128, zrow, 0)
        base = sid * rps
        nfull = rps // 128
        tail = rps % 128

        @pl.when(sid < nzw)
        def _init():
            for k in range(nfull):
                pltpu.sync_copy(z128, sums_sh.at[pl.ds(base + k * 128, 128)])
            if tail:
                lo = base + nfull * 128
                pltpu.sync_copy(z128.at[pl.ds(0, tail)],
                                sums_sh.at[pl.ds(lo, tail)])
        plsc.subcore_barrier()

        wid = sid * 2 + cid
        ntrip = (nchunk - wid + 31) // 32

        def lin_descr(q, s4):
            b0 = (wid + q * 32) * CHA
            return (
                pltpu.make_async_copy(src_hbm.at[pl.ds(b0, CHA)],
                                      sidx.at[s4], lsems[s4]),
                pltpu.make_async_copy(dst_hbm.at[pl.ds(b0, CHA)],
                                      didx.at[s4], lsems[s4]),
                pltpu.make_async_copy(g_hbm.at[pl.ds(b0, CHA)],
                                      gslot.at[s4], lsems[s4]),
                pltpu.make_async_copy(me_hbm.at[pl.ds(b0, CHA)],
                                      mes[s4 % 2], lsems[s4]),
            )

        def lin_issue(q, s4):
            @pl.when(q < ntrip)
            def _():
                for d in lin_descr(q, s4):
                    d.start()

        def lin_wait(q, s4):
            @pl.when(q < ntrip)
            def _():
                for d in lin_descr(q, s4):
                    d.wait()

        def gat_descr(q, s4, p):
            return pltpu.make_async_copy(hn_hbm.at[sidx.at[s4]],
                                         rws[p], gsems[p])

        def sct_descr(q, s4, p):
            return pltpu.make_async_copy(rws[p], sums_sh.at[didx.at[s4]],
                                         ssems[p])

        def step(q, s4, p):
            s4n = (s4 + 1) % 4
            s4nn = (s4 + 2) % 4
            pn = (p + 1) % 2

            @pl.when(q < ntrip)
            def _proc():
                gat_descr(q, s4, p).wait()
                me_v = mes[p]
                rw_v = rws[p]
                g_v = gslot

                def rowfn(gg, cc):
                    gvec = g_v[s4, pl.ds(gg * 16, 16)]

                    def row(r, c2):
                        i = gg * 16 + r
                        gs = gvec[jnp.full((16,), r, jnp.int32)]
                        for j in range(hd // 16):
                            sl = pl.ds(j * 16, 16)
                            rw_v[i, sl] = gs * rw_v[i, sl] + me_v[i, sl]
                        return c2
                    lax.fori_loop(0, 16, row, 0)
                    return cc
                lax.fori_loop(0, CHA // 16, rowfn, 0)
                sct_descr(q, s4, p).start(add=True)

            lin_issue(q + 2, s4nn)
            lin_wait(q + 1, s4n)

            @pl.when((q >= 1) & (q - 1 < ntrip))
            def _ws():
                sct_descr(q - 1, s4n, pn).wait()

            @pl.when(q + 1 < ntrip)
            def _g():
                gat_descr(q + 1, s4n, pn).start()

        lin_issue(0, 0)
        lin_issue(1, 1)
        lin_wait(0, 0)

        @pl.when(0 < ntrip)
        def _g0():
            gat_descr(0, 0, 0).start()

        def quad(i, c):
            q = i * 4
            step(q, 0, 0)
            step(q + 1, 1, 1)
            step(q + 2, 2, 0)
            step(q + 3, 3, 1)
            return c
        lax.fori_loop(0, npass // 4, quad, 0)


        plsc.subcore_barrier()

        @pl.when(sid < nzw)
        def _drain():
            for k in range(nfull):
                lo = base + k * 128
                pltpu.sync_copy(sums_sh.at[pl.ds(lo, 128)], z128)
                pltpu.sync_copy(z128, sums_out.at[cid, pl.ds(lo, 128)])
            if tail:
                lo = base + nfull * 128
                pltpu.sync_copy(sums_sh.at[pl.ds(lo, tail)],
                                z128.at[pl.ds(0, tail)])
                pltpu.sync_copy(z128.at[pl.ds(0, tail)],
                                sums_out.at[cid, pl.ds(lo, tail)])

    return kern(hn, me, g, src, dst)


def _sc_counts(dst, n, hd):
    e = dst.shape[0]
    nchunk = e // CH
    nzw = 10
    rps = n // nzw

    mesh = plsc.VectorSubcoreMesh(core_axis_name="c", subcore_axis_name="s")

    @functools.partial(
        pl.kernel, mesh=mesh,
        out_type=jax.ShapeDtypeStruct((2, n, hd), F32),
        scratch_types=[
            pltpu.VMEM((CH, hd), F32),
            pltpu.VMEM((CH,), jnp.int32),
            pltpu.VMEM((128, 128), F32),
            pltpu.VMEM_SHARED((n, hd), F32),
        ])
    def kern(dst_hbm, cnt_out, ones_v, dst_v, z128, cnt_sh):
        cid = lax.axis_index("c")
        sid = lax.axis_index("s")

        if True:
            def zrow(i, c):
                for j in range(8):
                    z128[i, pl.ds(j * 16, 16)] = jnp.zeros((16,), F32)
                    ones_v[i, pl.ds(j * 16, 16)] = jnp.full((16,), 1.0, F32)
                return c
            lax.fori_loop(0, 128, zrow, 0)
            base = sid * rps
            nfull = rps // 128
            tail = rps % 128

            @pl.when(sid < nzw)
            def _init():
                for k in range(nfull):
                    pltpu.sync_copy(z128,
                                    cnt_sh.at[pl.ds(base + k * 128, 128)])
                if tail:
                    lo = base + nfull * 128
                    pltpu.sync_copy(z128.at[pl.ds(0, tail)],
                                    cnt_sh.at[pl.ds(lo, tail)])
            plsc.subcore_barrier()

            wid = sid * 2 + cid
            ntrip = (nchunk - wid + 31) // 32

            def chunk(t, c):
                b0 = (wid + t * 32) * CH
                pltpu.sync_copy(dst_hbm.at[pl.ds(b0, CH)], dst_v)
                pltpu.sync_copy(ones_v, cnt_sh.at[dst_v], add=True)
                return c
            lax.fori_loop(0, ntrip, chunk, 0)

            plsc.subcore_barrier()

            @pl.when(sid < nzw)
            def _drain():
                for k in range(nfull):
                    lo = base + k * 128
                    pltpu.sync_copy(cnt_sh.at[pl.ds(lo, 128)], z128)
                    pltpu.sync_copy(z128, cnt_out.at[cid, pl.ds(lo, 128)])
                if tail:
                    lo = base + nfull * 128
                    pltpu.sync_copy(cnt_sh.at[pl.ds(lo, tail)],
                                    z128.at[pl.ds(0, tail)])
                    pltpu.sync_copy(z128.at[pl.ds(0, tail)],
                                    cnt_out.at[cid, pl.ds(lo, tail)])

    return kern(dst)


def kernel(x, edge_attr, params, edge_index):
    src = edge_index[0].astype(jnp.int32)
    dst = edge_index[1].astype(jnp.int32)

    h = _tc_mlp2(params["enc"], x)
    cnt_parts = _sc_counts(dst, x.shape[0], 128)
    for lp in params["layers"]:
        ee, g = _tc_edge(lp, edge_attr)
        hn = _tc_lin(lp["neigh"], h)
        sums_parts = _sc_aggregate(hn, ee, g, src, dst)
        h = _tc_update(lp, h, sums_parts, cnt_parts)
    return _tc_head(params["head"], params["scale"], h)
